# baseline (speedup 1.0000x reference)
import functools

import jax
import jax.numpy as jnp
from jax import lax
from jax.experimental import pallas as pl
from jax.experimental.pallas import tpu as pltpu
from jax.experimental.pallas import tpu_sc as plsc

N = 10000
E = 320000
D = 128
LAT = 64
NG = 64

NC = 2
NS = 16
NW = NC * NS
EPW = E // NW
CH = 128
SC_EPW = 10240
SC_NCH = SC_EPW // CH
PADE = NW * SC_EPW
PAD_ROWS = 128
ACC_ROWS = N + PAD_ROWS
NBUF = 2
DR = 1000
NDR = N // DR
DEGW = 8

NB = 10
BR = N // NB


def _deg_body(dst_hbm, out_hbm, didx, deg_v, sem):
    c = lax.axis_index("c")
    s = lax.axis_index("s")
    w = c * NS + s
    pltpu.sync_copy(dst_hbm.at[pl.ds(w * EPW, EPW)], didx)

    def zbody(j, carry):
        deg_v[pl.ds(j * 16, 16)] = jnp.zeros((16,), jnp.float32)
        return carry

    lax.fori_loop(0, N // 16, zbody, 0)

    ones = jnp.ones((16,), jnp.float32)

    def body(j, carry):
        idx = didx[pl.ds(j * 16, 16)]
        plsc.addupdate_scatter(deg_v, [idx], ones)
        return carry

    lax.fori_loop(0, EPW // 16, body, 0)
    pltpu.sync_copy(deg_v, out_hbm.at[pl.ds(w * N, N)])
    del sem


@functools.cache
def _deg_call():
    mesh = plsc.VectorSubcoreMesh(core_axis_name="c", subcore_axis_name="s")
    return pl.kernel(
        _deg_body,
        out_type=jax.ShapeDtypeStruct((NW * N,), jnp.float32),
        mesh=mesh,
        compiler_params=pltpu.CompilerParams(needs_layout_passes=False),
        scratch_types=[
            pltpu.VMEM((EPW,), jnp.int32),
            pltpu.VMEM((N,), jnp.float32),
            pltpu.SemaphoreType.DMA,
        ],
    )


def _scatter_body(g_hbm, src_hbm, dst_hbm, zeros_hbm, out_hbm,
                  sidx_all, didx_buf, bufs, acc, isem, gsem, ssem):
    c = lax.axis_index("c")
    s = lax.axis_index("s")
    w = c * NS + s

    @pl.when(s < NDR)
    def _():
        pltpu.sync_copy(zeros_hbm, acc.at[pl.ds(s * DR, DR)])

    @pl.when(s == NDR)
    def _():
        pltpu.sync_copy(zeros_hbm.at[pl.ds(0, PAD_ROWS)],
                        acc.at[pl.ds(N, PAD_ROWS)])

    pltpu.sync_copy(src_hbm.at[w], sidx_all)
    plsc.subcore_barrier()

    def fire(j, p):
        pltpu.async_copy(dst_hbm.at[pl.ds(w * SC_EPW + j * CH, CH)],
                         didx_buf.at[p], isem.at[p])
        pltpu.async_copy(g_hbm.at[sidx_all.at[j]], bufs.at[p], gsem.at[p])

    def wait_and_scatter(j, p):
        pltpu.make_async_copy(dst_hbm.at[pl.ds(0, CH)],
                              didx_buf.at[p], isem.at[p]).wait()
        pltpu.make_async_copy(g_hbm.at[sidx_all.at[j]],
                              bufs.at[p], gsem.at[p]).wait()
        return pltpu.async_copy(bufs.at[p], acc.at[didx_buf.at[p]],
                                ssem.at[p], add=True)

    for p in range(NBUF):
        fire(p, p)

    NITER = SC_NCH // NBUF

    def body(i, carry):
        j0 = i * NBUF
        ss = [wait_and_scatter(j0 + k, k) for k in range(NBUF)]
        for k in range(NBUF):
            ss[k].wait()
            jn = j0 + k + NBUF

            @pl.when(jn < SC_NCH)
            def _():
                fire(jn, k)

        return carry

    lax.fori_loop(0, NITER, body, 0)
    plsc.subcore_barrier()

    @pl.when(s < NDR)
    def _():
        pltpu.sync_copy(acc.at[pl.ds(s * DR, DR)],
                        out_hbm.at[c, pl.ds(s * DR, DR)])


@functools.cache
def _scatter_call():
    mesh = plsc.VectorSubcoreMesh(core_axis_name="c", subcore_axis_name="s")
    return pl.kernel(
        _scatter_body,
        out_type=jax.ShapeDtypeStruct((NC, N, D), jnp.float32),
        mesh=mesh,
        scratch_types=[
            pltpu.VMEM((SC_NCH, CH), jnp.int32),
            pltpu.VMEM((NBUF, CH), jnp.int32),
            pltpu.VMEM((NBUF, CH, D), jnp.float32),
            pltpu.VMEM_SHARED((ACC_ROWS, D), jnp.float32),
            pltpu.SemaphoreType.DMA((NBUF,)),
            pltpu.SemaphoreType.DMA((NBUF,)),
            pltpu.SemaphoreType.DMA((NBUF,)),
        ],
    )


def _dinv_block(degp_ref):
    deg = 1.0 + jnp.sum(degp_ref[0], axis=1)[:, None]
    return lax.rsqrt(jnp.maximum(deg, 1.0))


def _tc1_body(x_ref, w1_ref, degp_ref, g_ref):
    dinv = _dinv_block(degp_ref)
    h = jnp.dot(x_ref[...], w1_ref[...], preferred_element_type=jnp.float32)
    g_ref[...] = h * dinv


_tc1_call = pl.pallas_call(
    _tc1_body,
    grid=(NB,),
    in_specs=[
        pl.BlockSpec((BR, D), lambda i: (i, 0)),
        pl.BlockSpec((D, D), lambda i: (0, 0)),
        pl.BlockSpec((1, BR, NW), lambda i: (i, 0, 0)),
    ],
    out_specs=pl.BlockSpec((BR, D), lambda i: (i, 0)),
    out_shape=jax.ShapeDtypeStruct((N, D), jnp.float32),
)


def _tc2_body(accp_ref, g1_ref, degp_ref, b1_ref, w2_ref, g2_ref):
    dinv = _dinv_block(degp_ref)
    acc = accp_ref[0] + accp_ref[1] + g1_ref[...]
    h = jnp.maximum(acc * dinv + b1_ref[...], 0.0)
    g2_ref[...] = jnp.dot(h, w2_ref[...],
                          preferred_element_type=jnp.float32) * dinv


_tc2_call = pl.pallas_call(
    _tc2_body,
    grid=(NB,),
    in_specs=[
        pl.BlockSpec((NC, BR, D), lambda i: (0, i, 0)),
        pl.BlockSpec((BR, D), lambda i: (i, 0)),
        pl.BlockSpec((1, BR, NW), lambda i: (i, 0, 0)),
        pl.BlockSpec((1, D), lambda i: (0, 0)),
        pl.BlockSpec((D, D), lambda i: (0, 0)),
    ],
    out_specs=pl.BlockSpec((BR, D), lambda i: (i, 0)),
    out_shape=jax.ShapeDtypeStruct((N, D), jnp.float32),
)


def _tc3_body(accp_ref, g2_ref, degp_ref, b2_ref, batch_ref,
              wmu_ref, bmu_ref, wlv_ref, blv_ref,
              mu_ref, lv_ref, sums, cnts):
    i = pl.program_id(0)
    dinv = _dinv_block(degp_ref)
    acc = accp_ref[0] + accp_ref[1] + g2_ref[...]
    h = jnp.maximum(acc * dinv + b2_ref[...], 0.0)
    b = batch_ref[0, 0, :]
    gid = lax.broadcasted_iota(jnp.int32, (BR, NG), 1)
    oh = (b[:, None] == gid).astype(jnp.float32)

    @pl.when(i == 0)
    def _():
        sums[...] = jnp.zeros_like(sums)
        cnts[...] = jnp.zeros_like(cnts)

    dn = (((0,), (0,)), ((), ()))
    sums[...] += lax.dot_general(oh, h, dn,
                                 preferred_element_type=jnp.float32)
    cnts[...] += lax.dot_general(oh, jnp.ones((BR, D), jnp.float32), dn,
                                 preferred_element_type=jnp.float32)

    @pl.when(i == NB - 1)
    def _():
        means = sums[...] / jnp.maximum(cnts[...], 1.0)
        mu_ref[...] = jnp.dot(means, wmu_ref[...],
                              preferred_element_type=jnp.float32) + bmu_ref[...]
        lv_ref[...] = jnp.dot(means, wlv_ref[...],
                              preferred_element_type=jnp.float32) + blv_ref[...]


_tc3_call = pl.pallas_call(
    _tc3_body,
    grid=(NB,),
    in_specs=[
        pl.BlockSpec((NC, BR, D), lambda i: (0, i, 0)),
        pl.BlockSpec((BR, D), lambda i: (i, 0)),
        pl.BlockSpec((1, BR, NW), lambda i: (i, 0, 0)),
        pl.BlockSpec((1, D), lambda i: (0, 0)),
        pl.BlockSpec((1, 1, BR), lambda i: (i, 0, 0)),
        pl.BlockSpec((D, LAT), lambda i: (0, 0)),
        pl.BlockSpec((1, LAT), lambda i: (0, 0)),
        pl.BlockSpec((D, LAT), lambda i: (0, 0)),
        pl.BlockSpec((1, LAT), lambda i: (0, 0)),
    ],
    out_specs=[
        pl.BlockSpec((NG, LAT), lambda i: (0, 0)),
        pl.BlockSpec((NG, LAT), lambda i: (0, 0)),
    ],
    out_shape=[
        jax.ShapeDtypeStruct((NG, LAT), jnp.float32),
        jax.ShapeDtypeStruct((NG, LAT), jnp.float32),
    ],
    scratch_shapes=[
        pltpu.VMEM((NG, D), jnp.float32),
        pltpu.VMEM((NG, D), jnp.float32),
    ],
)


def kernel(x, edge_index, batch, W1, b1, W2, b2, Wmu, bmu, Wlv, blv):
    src_flat = edge_index[0]
    dst_flat = edge_index[1]
    zeros_row = jnp.zeros((DR, D), jnp.float32)
    batch3 = batch.reshape(NB, 1, BR)

    pad_n = PADE - E
    src_pad = jnp.concatenate(
        [src_flat, jnp.zeros((pad_n,), jnp.int32)])
    dst_pad = jnp.concatenate(
        [dst_flat,
         N + (jnp.arange(pad_n, dtype=jnp.int32) % PAD_ROWS)])
    src_r = src_pad.reshape(NW, SC_NCH, CH)
    degp = _deg_call()(dst_flat).reshape(NW, NB, BR).transpose(1, 2, 0)
    g1 = _tc1_call(x, W1, degp)
    acc1 = _scatter_call()(g1, src_r, dst_pad, zeros_row)
    g2 = _tc2_call(acc1, g1, degp, b1.reshape(1, D), W2)
    acc2 = _scatter_call()(g2, src_r, dst_pad, zeros_row)
    mu, lv = _tc3_call(acc2, g2, degp, b2.reshape(1, D), batch3,
                       Wmu, bmu.reshape(1, LAT), Wlv, blv.reshape(1, LAT))
    return (mu, lv)


# speedup vs baseline: 2.9268x; 2.9268x over previous
import functools

import jax
import jax.numpy as jnp
from jax import lax
from jax.experimental import pallas as pl
from jax.experimental.pallas import tpu as pltpu
from jax.experimental.pallas import tpu_sc as plsc

N = 10000
E = 320000
D = 128
LAT = 64
NG = 64

NC = 2
NS = 16
NW = NC * NS
EPW = E // NW
CH = 80
SC_NCH = EPW // CH
NBUF = 3
DR = 1000
NDR = N // DR
DEGW = 8

NB = 10
BR = N // NB


def _deg_body(dst_hbm, out_hbm, didx, deg_v, sem):
    c = lax.axis_index("c")
    s = lax.axis_index("s")
    w = c * NS + s
    pltpu.sync_copy(dst_hbm.at[pl.ds(w * EPW, EPW)], didx)

    def zbody(j, carry):
        deg_v[pl.ds(j * 16, 16)] = jnp.zeros((16,), jnp.float32)
        return carry

    lax.fori_loop(0, N // 16, zbody, 0)

    ones = jnp.ones((16,), jnp.float32)

    def body(j, carry):
        idx = didx[pl.ds(j * 16, 16)]
        plsc.addupdate_scatter(deg_v, [idx], ones)
        return carry

    lax.fori_loop(0, EPW // 16, body, 0)
    pltpu.sync_copy(deg_v, out_hbm.at[pl.ds(w * N, N)])
    del sem


@functools.cache
def _deg_call():
    mesh = plsc.VectorSubcoreMesh(core_axis_name="c", subcore_axis_name="s")
    return pl.kernel(
        _deg_body,
        out_type=jax.ShapeDtypeStruct((NW * N,), jnp.float32),
        mesh=mesh,
        compiler_params=pltpu.CompilerParams(needs_layout_passes=False),
        scratch_types=[
            pltpu.VMEM((EPW,), jnp.int32),
            pltpu.VMEM((N,), jnp.float32),
            pltpu.SemaphoreType.DMA,
        ],
    )


def _scatter_body(g_hbm, src_hbm, dst_hbm, zeros_hbm, out_hbm,
                  sidx_all, didx_buf, bufs, acc, isem, gsem, ssem):
    c = lax.axis_index("c")
    s = lax.axis_index("s")
    w = c * NS + s

    @pl.when(s < NDR)
    def _():
        pltpu.sync_copy(zeros_hbm, acc.at[pl.ds(s * DR, DR)])

    pltpu.sync_copy(src_hbm.at[w], sidx_all)
    plsc.subcore_barrier()

    def fire(j, p):
        pltpu.async_copy(dst_hbm.at[pl.ds(w * EPW + j * CH, CH)],
                         didx_buf.at[p], isem.at[p])
        pltpu.async_copy(g_hbm.at[sidx_all.at[j]], bufs.at[p], gsem.at[p])

    def wait_and_scatter(j, p):
        pltpu.make_async_copy(dst_hbm.at[pl.ds(0, CH)],
                              didx_buf.at[p], isem.at[p]).wait()
        pltpu.make_async_copy(g_hbm.at[sidx_all.at[j]],
                              bufs.at[p], gsem.at[p]).wait()
        return pltpu.async_copy(bufs.at[p], acc.at[didx_buf.at[p]],
                                ssem.at[p], add=True)

    for p in range(NBUF):
        fire(p, p)

    NITER = SC_NCH // NBUF

    def body(i, carry):
        j0 = i * NBUF
        ss = [wait_and_scatter(j0 + k, k) for k in range(NBUF)]
        for k in range(NBUF):
            ss[k].wait()
            jn = j0 + k + NBUF

            @pl.when(jn < SC_NCH)
            def _():
                fire(jn, k)

        return carry

    lax.fori_loop(0, NITER, body, 0)
    tail = [(j, j % NBUF) for j in range(NITER * NBUF, SC_NCH)]
    ts = [wait_and_scatter(j, p) for j, p in tail]
    for t in ts:
        t.wait()
    plsc.subcore_barrier()

    @pl.when(s < NDR)
    def _():
        pltpu.sync_copy(acc.at[pl.ds(s * DR, DR)],
                        out_hbm.at[c, pl.ds(s * DR, DR)])


@functools.cache
def _scatter_call():
    mesh = plsc.VectorSubcoreMesh(core_axis_name="c", subcore_axis_name="s")
    return pl.kernel(
        _scatter_body,
        out_type=jax.ShapeDtypeStruct((NC, N, D), jnp.float32),
        mesh=mesh,
        scratch_types=[
            pltpu.VMEM((SC_NCH, CH), jnp.int32),
            pltpu.VMEM((NBUF, CH), jnp.int32),
            pltpu.VMEM((NBUF, CH, D), jnp.float32),
            pltpu.VMEM_SHARED((N, D), jnp.float32),
            pltpu.SemaphoreType.DMA((NBUF,)),
            pltpu.SemaphoreType.DMA((NBUF,)),
            pltpu.SemaphoreType.DMA((NBUF,)),
        ],
    )


def _dinv_block(degp_ref):
    deg = 1.0 + jnp.sum(degp_ref[0], axis=1)[:, None]
    return lax.rsqrt(jnp.maximum(deg, 1.0))


def _tc1_body(x_ref, w1_ref, degp_ref, g_ref):
    dinv = _dinv_block(degp_ref)
    h = jnp.dot(x_ref[...], w1_ref[...], preferred_element_type=jnp.float32)
    g_ref[...] = h * dinv


_tc1_call = pl.pallas_call(
    _tc1_body,
    grid=(NB,),
    in_specs=[
        pl.BlockSpec((BR, D), lambda i: (i, 0)),
        pl.BlockSpec((D, D), lambda i: (0, 0)),
        pl.BlockSpec((1, BR, NW), lambda i: (i, 0, 0)),
    ],
    out_specs=pl.BlockSpec((BR, D), lambda i: (i, 0)),
    out_shape=jax.ShapeDtypeStruct((N, D), jnp.float32),
)


def _tc2_body(accp_ref, g1_ref, degp_ref, b1_ref, w2_ref, g2_ref):
    dinv = _dinv_block(degp_ref)
    acc = accp_ref[0] + accp_ref[1] + g1_ref[...]
    h = jnp.maximum(acc * dinv + b1_ref[...], 0.0)
    g2_ref[...] = jnp.dot(h, w2_ref[...],
                          preferred_element_type=jnp.float32) * dinv


_tc2_call = pl.pallas_call(
    _tc2_body,
    grid=(NB,),
    in_specs=[
        pl.BlockSpec((NC, BR, D), lambda i: (0, i, 0)),
        pl.BlockSpec((BR, D), lambda i: (i, 0)),
        pl.BlockSpec((1, BR, NW), lambda i: (i, 0, 0)),
        pl.BlockSpec((1, D), lambda i: (0, 0)),
        pl.BlockSpec((D, D), lambda i: (0, 0)),
    ],
    out_specs=pl.BlockSpec((BR, D), lambda i: (i, 0)),
    out_shape=jax.ShapeDtypeStruct((N, D), jnp.float32),
)


def _tc3_body(accp_ref, g2_ref, degp_ref, b2_ref, batch_ref,
              wmu_ref, bmu_ref, wlv_ref, blv_ref,
              mu_ref, lv_ref, sums, cnts):
    i = pl.program_id(0)
    dinv = _dinv_block(degp_ref)
    acc = accp_ref[0] + accp_ref[1] + g2_ref[...]
    h = jnp.maximum(acc * dinv + b2_ref[...], 0.0)
    b = batch_ref[0, 0, :]
    gid = lax.broadcasted_iota(jnp.int32, (BR, NG), 1)
    oh = (b[:, None] == gid).astype(jnp.float32)

    @pl.when(i == 0)
    def _():
        sums[...] = jnp.zeros_like(sums)
        cnts[...] = jnp.zeros_like(cnts)

    dn = (((0,), (0,)), ((), ()))
    sums[...] += lax.dot_general(oh, h, dn,
                                 preferred_element_type=jnp.float32)
    cnts[...] += lax.dot_general(oh, jnp.ones((BR, D), jnp.float32), dn,
                                 preferred_element_type=jnp.float32)

    @pl.when(i == NB - 1)
    def _():
        means = sums[...] / jnp.maximum(cnts[...], 1.0)
        mu_ref[...] = jnp.dot(means, wmu_ref[...],
                              preferred_element_type=jnp.float32) + bmu_ref[...]
        lv_ref[...] = jnp.dot(means, wlv_ref[...],
                              preferred_element_type=jnp.float32) + blv_ref[...]


_tc3_call = pl.pallas_call(
    _tc3_body,
    grid=(NB,),
    in_specs=[
        pl.BlockSpec((NC, BR, D), lambda i: (0, i, 0)),
        pl.BlockSpec((BR, D), lambda i: (i, 0)),
        pl.BlockSpec((1, BR, NW), lambda i: (i, 0, 0)),
        pl.BlockSpec((1, D), lambda i: (0, 0)),
        pl.BlockSpec((1, 1, BR), lambda i: (i, 0, 0)),
        pl.BlockSpec((D, LAT), lambda i: (0, 0)),
        pl.BlockSpec((1, LAT), lambda i: (0, 0)),
        pl.BlockSpec((D, LAT), lambda i: (0, 0)),
        pl.BlockSpec((1, LAT), lambda i: (0, 0)),
    ],
    out_specs=[
        pl.BlockSpec((NG, LAT), lambda i: (0, 0)),
        pl.BlockSpec((NG, LAT), lambda i: (0, 0)),
    ],
    out_shape=[
        jax.ShapeDtypeStruct((NG, LAT), jnp.float32),
        jax.ShapeDtypeStruct((NG, LAT), jnp.float32),
    ],
    scratch_shapes=[
        pltpu.VMEM((NG, D), jnp.float32),
        pltpu.VMEM((NG, D), jnp.float32),
    ],
)


def kernel(x, edge_index, batch, W1, b1, W2, b2, Wmu, bmu, Wlv, blv):
    src_flat = edge_index[0]
    dst_flat = edge_index[1]
    zeros_row = jnp.zeros((DR, D), jnp.float32)
    batch3 = batch.reshape(NB, 1, BR)

    src_r = src_flat.reshape(NW, SC_NCH, CH)
    degp = _deg_call()(dst_flat).reshape(NW, NB, BR).transpose(1, 2, 0)
    g1 = _tc1_call(x, W1, degp)
    acc1 = _scatter_call()(g1, src_r, dst_flat, zeros_row)
    g2 = _tc2_call(acc1, g1, degp, b1.reshape(1, D), W2)
    acc2 = _scatter_call()(g2, src_r, dst_flat, zeros_row)
    mu, lv = _tc3_call(acc2, g2, degp, b2.reshape(1, D), batch3,
                       Wmu, bmu.reshape(1, LAT), Wlv, blv.reshape(1, LAT))
    return (mu, lv)


# speedup vs baseline: 2.9651x; 1.0131x over previous
import functools

import jax
import jax.numpy as jnp
from jax import lax
from jax.experimental import pallas as pl
from jax.experimental.pallas import tpu as pltpu
from jax.experimental.pallas import tpu_sc as plsc

N = 10000
E = 320000
D = 128
LAT = 64
NG = 64

NC = 2
NS = 16
NW = NC * NS
EPW = E // NW
CH = 80
SC_NCH = EPW // CH
NBUF = 3
DR = 1000
NDR = N // DR
DEGW = 8

NB = 10
BR = N // NB


def _deg_body(dst_hbm, out_hbm, didx, deg_v, sem):
    c = lax.axis_index("c")
    s = lax.axis_index("s")
    w = c * NS + s
    pltpu.sync_copy(dst_hbm.at[pl.ds(w * EPW, EPW)], didx)

    def zbody(j, carry):
        deg_v[pl.ds(j * 16, 16)] = jnp.zeros((16,), jnp.float32)
        return carry

    lax.fori_loop(0, N // 16, zbody, 0)

    ones = jnp.ones((16,), jnp.float32)

    def body(j, carry):
        idx = didx[pl.ds(j * 16, 16)]
        plsc.addupdate_scatter(deg_v, [idx], ones)
        return carry

    lax.fori_loop(0, EPW // 16, body, 0)
    pltpu.sync_copy(deg_v, out_hbm.at[pl.ds(w * N, N)])
    del sem


@functools.cache
def _deg_call():
    mesh = plsc.VectorSubcoreMesh(core_axis_name="c", subcore_axis_name="s")
    return pl.kernel(
        _deg_body,
        out_type=jax.ShapeDtypeStruct((NW * N,), jnp.float32),
        mesh=mesh,
        compiler_params=pltpu.CompilerParams(needs_layout_passes=False),
        scratch_types=[
            pltpu.VMEM((EPW,), jnp.int32),
            pltpu.VMEM((N,), jnp.float32),
            pltpu.SemaphoreType.DMA,
        ],
    )


def _scatter_body(g_hbm, src_hbm, dst_hbm, zeros_hbm, out_hbm,
                  sidx_all, didx_buf, bufs, acc, isem, gsem, ssem, psem):
    c = lax.axis_index("c")
    s = lax.axis_index("s")
    w = c * NS + s

    @pl.when(s < NDR)
    def _():
        pltpu.async_copy(zeros_hbm, acc.at[pl.ds(s * DR, DR)], psem.at[0])

    pltpu.async_copy(src_hbm.at[w], sidx_all, psem.at[1])

    @pl.when(s < NDR)
    def _():
        pltpu.make_async_copy(zeros_hbm, acc.at[pl.ds(s * DR, DR)],
                              psem.at[0]).wait()

    pltpu.make_async_copy(src_hbm.at[w], sidx_all, psem.at[1]).wait()
    plsc.subcore_barrier()

    def fire(j, p):
        pltpu.async_copy(dst_hbm.at[pl.ds(w * EPW + j * CH, CH)],
                         didx_buf.at[p], isem.at[p])
        pltpu.async_copy(g_hbm.at[sidx_all.at[j]], bufs.at[p], gsem.at[p])

    def wait_and_scatter(j, p):
        pltpu.make_async_copy(dst_hbm.at[pl.ds(0, CH)],
                              didx_buf.at[p], isem.at[p]).wait()
        pltpu.make_async_copy(g_hbm.at[sidx_all.at[j]],
                              bufs.at[p], gsem.at[p]).wait()
        return pltpu.async_copy(bufs.at[p], acc.at[didx_buf.at[p]],
                                ssem.at[p], add=True)

    for p in range(NBUF):
        fire(p, p)

    NITER = SC_NCH // NBUF

    def body(i, carry):
        j0 = i * NBUF
        ss = [wait_and_scatter(j0 + k, k) for k in range(NBUF)]
        for k in range(NBUF):
            ss[k].wait()
            jn = j0 + k + NBUF

            @pl.when(jn < SC_NCH)
            def _():
                fire(jn, k)

        return carry

    lax.fori_loop(0, NITER, body, 0)
    tail = [(j, j % NBUF) for j in range(NITER * NBUF, SC_NCH)]
    ts = [wait_and_scatter(j, p) for j, p in tail]
    for t in ts:
        t.wait()
    plsc.subcore_barrier()

    @pl.when(s < NDR)
    def _():
        pltpu.sync_copy(acc.at[pl.ds(s * DR, DR)],
                        out_hbm.at[c, pl.ds(s * DR, DR)])


@functools.cache
def _scatter_call():
    mesh = plsc.VectorSubcoreMesh(core_axis_name="c", subcore_axis_name="s")
    return pl.kernel(
        _scatter_body,
        out_type=jax.ShapeDtypeStruct((NC, N, D), jnp.float32),
        mesh=mesh,
        scratch_types=[
            pltpu.VMEM((SC_NCH, CH), jnp.int32),
            pltpu.VMEM((NBUF, CH), jnp.int32),
            pltpu.VMEM((NBUF, CH, D), jnp.float32),
            pltpu.VMEM_SHARED((N, D), jnp.float32),
            pltpu.SemaphoreType.DMA((NBUF,)),
            pltpu.SemaphoreType.DMA((NBUF,)),
            pltpu.SemaphoreType.DMA((NBUF,)),
            pltpu.SemaphoreType.DMA((2,)),
        ],
    )


def _dinv_block(degp_ref):
    deg = 1.0 + jnp.sum(degp_ref[0], axis=1)[:, None]
    return lax.rsqrt(jnp.maximum(deg, 1.0))


def _tc1_body(x_ref, w1_ref, degp_ref, g_ref):
    dinv = _dinv_block(degp_ref)
    h = jnp.dot(x_ref[...], w1_ref[...], preferred_element_type=jnp.float32)
    g_ref[...] = h * dinv


_tc1_call = pl.pallas_call(
    _tc1_body,
    grid=(NB,),
    in_specs=[
        pl.BlockSpec((BR, D), lambda i: (i, 0)),
        pl.BlockSpec((D, D), lambda i: (0, 0)),
        pl.BlockSpec((1, BR, NW), lambda i: (i, 0, 0)),
    ],
    out_specs=pl.BlockSpec((BR, D), lambda i: (i, 0)),
    out_shape=jax.ShapeDtypeStruct((N, D), jnp.float32),
)


def _tc2_body(accp_ref, g1_ref, degp_ref, b1_ref, w2_ref, g2_ref):
    dinv = _dinv_block(degp_ref)
    acc = accp_ref[0] + accp_ref[1] + g1_ref[...]
    h = jnp.maximum(acc * dinv + b1_ref[...], 0.0)
    g2_ref[...] = jnp.dot(h, w2_ref[...],
                          preferred_element_type=jnp.float32) * dinv


_tc2_call = pl.pallas_call(
    _tc2_body,
    grid=(NB,),
    in_specs=[
        pl.BlockSpec((NC, BR, D), lambda i: (0, i, 0)),
        pl.BlockSpec((BR, D), lambda i: (i, 0)),
        pl.BlockSpec((1, BR, NW), lambda i: (i, 0, 0)),
        pl.BlockSpec((1, D), lambda i: (0, 0)),
        pl.BlockSpec((D, D), lambda i: (0, 0)),
    ],
    out_specs=pl.BlockSpec((BR, D), lambda i: (i, 0)),
    out_shape=jax.ShapeDtypeStruct((N, D), jnp.float32),
)


def _tc3_body(accp_ref, g2_ref, degp_ref, b2_ref, batch_ref,
              wmu_ref, bmu_ref, wlv_ref, blv_ref,
              mu_ref, lv_ref, sums, cnts):
    i = pl.program_id(0)
    dinv = _dinv_block(degp_ref)
    acc = accp_ref[0] + accp_ref[1] + g2_ref[...]
    h = jnp.maximum(acc * dinv + b2_ref[...], 0.0)
    b = batch_ref[0, 0, :]
    gid = lax.broadcasted_iota(jnp.int32, (BR, NG), 1)
    oh = (b[:, None] == gid).astype(jnp.float32)

    @pl.when(i == 0)
    def _():
        sums[...] = jnp.zeros_like(sums)
        cnts[...] = jnp.zeros_like(cnts)

    dn = (((0,), (0,)), ((), ()))
    sums[...] += lax.dot_general(oh, h, dn,
                                 preferred_element_type=jnp.float32)
    cnts[...] += lax.dot_general(oh, jnp.ones((BR, D), jnp.float32), dn,
                                 preferred_element_type=jnp.float32)

    @pl.when(i == NB - 1)
    def _():
        means = sums[...] / jnp.maximum(cnts[...], 1.0)
        mu_ref[...] = jnp.dot(means, wmu_ref[...],
                              preferred_element_type=jnp.float32) + bmu_ref[...]
        lv_ref[...] = jnp.dot(means, wlv_ref[...],
                              preferred_element_type=jnp.float32) + blv_ref[...]


_tc3_call = pl.pallas_call(
    _tc3_body,
    grid=(NB,),
    in_specs=[
        pl.BlockSpec((NC, BR, D), lambda i: (0, i, 0)),
        pl.BlockSpec((BR, D), lambda i: (i, 0)),
        pl.BlockSpec((1, BR, NW), lambda i: (i, 0, 0)),
        pl.BlockSpec((1, D), lambda i: (0, 0)),
        pl.BlockSpec((1, 1, BR), lambda i: (i, 0, 0)),
        pl.BlockSpec((D, LAT), lambda i: (0, 0)),
        pl.BlockSpec((1, LAT), lambda i: (0, 0)),
        pl.BlockSpec((D, LAT), lambda i: (0, 0)),
        pl.BlockSpec((1, LAT), lambda i: (0, 0)),
    ],
    out_specs=[
        pl.BlockSpec((NG, LAT), lambda i: (0, 0)),
        pl.BlockSpec((NG, LAT), lambda i: (0, 0)),
    ],
    out_shape=[
        jax.ShapeDtypeStruct((NG, LAT), jnp.float32),
        jax.ShapeDtypeStruct((NG, LAT), jnp.float32),
    ],
    scratch_shapes=[
        pltpu.VMEM((NG, D), jnp.float32),
        pltpu.VMEM((NG, D), jnp.float32),
    ],
)


def kernel(x, edge_index, batch, W1, b1, W2, b2, Wmu, bmu, Wlv, blv):
    src_flat = edge_index[0]
    dst_flat = edge_index[1]
    zeros_row = jnp.zeros((DR, D), jnp.float32)
    batch3 = batch.reshape(NB, 1, BR)

    src_r = src_flat.reshape(NW, SC_NCH, CH)
    degp = _deg_call()(dst_flat).reshape(NW, NB, BR).transpose(1, 2, 0)
    g1 = _tc1_call(x, W1, degp)
    acc1 = _scatter_call()(g1, src_r, dst_flat, zeros_row)
    g2 = _tc2_call(acc1, g1, degp, b1.reshape(1, D), W2)
    acc2 = _scatter_call()(g2, src_r, dst_flat, zeros_row)
    mu, lv = _tc3_call(acc2, g2, degp, b2.reshape(1, D), batch3,
                       Wmu, bmu.reshape(1, LAT), Wlv, blv.reshape(1, LAT))
    return (mu, lv)
